# baseline (speedup 1.0000x reference)
import jax
import jax.numpy as jnp
from jax import lax
from jax.experimental import pallas as pl
from jax.experimental.pallas import tpu as pltpu
from jax.experimental.pallas import tpu_sc as plsc

_BM = 2048

_NC = 2
_NS = 16
_NW = _NC * _NS


def _gate_body(x_ref, w1_ref, b1_ref, w2_ref, b2_ref, probs_ref):
    h = jnp.dot(x_ref[...], w1_ref[...], preferred_element_type=jnp.float32)
    h = h + b1_ref[...]
    h = h * jax.nn.sigmoid(h)
    logits = jnp.dot(h, w2_ref[...], preferred_element_type=jnp.float32)
    logits = logits + b2_ref[...]
    m = jnp.max(logits, axis=1, keepdims=True)
    e = jnp.exp(logits - m)
    probs_ref[...] = e / jnp.sum(e, axis=1, keepdims=True)


def _topk_body(probs_hbm, sparse_hbm, idx_hbm, probs_v, sparse_v, idx_v):
    tpw = probs_v.shape[0]
    wid = lax.axis_index("s") * _NC + lax.axis_index("c")
    base = wid * tpw
    pltpu.sync_copy(probs_hbm.at[pl.ds(base, tpw)], probs_v)
    lanes = lax.iota(jnp.int32, 16)

    top2 = lanes < 2

    @plsc.parallel_loop(0, tpw, unroll=8)
    def _body(t):
        p = probs_v[t]
        sk, sv = plsc.sort_key_val(p, lanes, descending=True)
        row = lanes * 0 + t
        sparse_v[t] = jnp.zeros((16,), jnp.float32)
        plsc.store_scatter(sparse_v, [row, sv], sk, mask=top2)
        plsc.store_scatter(idx_v, [row, lanes], sv, mask=top2)

    pltpu.sync_copy(sparse_v, sparse_hbm.at[pl.ds(base, tpw)])
    pltpu.sync_copy(idx_v, idx_hbm.at[pl.ds(base, tpw)])


@jax.jit
def kernel(x, W1, b1, W2, b2):
    n_tokens, hidden = x.shape
    n_mod = W2.shape[1]
    n_hid = W1.shape[1]
    probs = pl.pallas_call(
        _gate_body,
        grid=(n_tokens // _BM,),
        in_specs=[
            pl.BlockSpec((_BM, hidden), lambda i: (i, 0)),
            pl.BlockSpec((hidden, n_hid), lambda i: (0, 0)),
            pl.BlockSpec((n_hid,), lambda i: (0,)),
            pl.BlockSpec((n_hid, n_mod), lambda i: (0, 0)),
            pl.BlockSpec((n_mod,), lambda i: (0,)),
        ],
        out_specs=pl.BlockSpec((_BM, n_mod), lambda i: (i, 0)),
        out_shape=jax.ShapeDtypeStruct((n_tokens, n_mod), jnp.float32),
    )(x, W1, b1, W2, b2)

    tpw = n_tokens // _NW
    sparse, idx = pl.kernel(
        _topk_body,
        out_type=[
            jax.ShapeDtypeStruct((n_tokens, n_mod), jnp.float32),
            jax.ShapeDtypeStruct((n_tokens, 2), jnp.int32),
        ],
        mesh=plsc.VectorSubcoreMesh(
            core_axis_name="c", subcore_axis_name="s",
            num_cores=_NC, num_subcores=_NS,
        ),
        compiler_params=pltpu.CompilerParams(needs_layout_passes=False, skip_device_barrier=True),
        scratch_types=[
            pltpu.VMEM((tpw, n_mod), jnp.float32),
            pltpu.VMEM((tpw, n_mod), jnp.float32),
            pltpu.VMEM((tpw, 2), jnp.int32),
        ],
    )(probs)
    return (sparse, idx)


# speedup vs baseline: 1.0725x; 1.0725x over previous
import jax
import jax.numpy as jnp
from jax import lax
from jax.experimental import pallas as pl
from jax.experimental.pallas import tpu as pltpu
from jax.experimental.pallas import tpu_sc as plsc

_BM = 1024

_NC = 2
_NS = 16
_NW = _NC * _NS


def _gate_body(x_ref, w1_ref, b1_ref, w2_ref, b2_ref, probs_ref):
    h = jnp.dot(x_ref[...], w1_ref[...], preferred_element_type=jnp.float32)
    h = h + b1_ref[...]
    h = h * jax.nn.sigmoid(h)
    logits = jnp.dot(h, w2_ref[...], preferred_element_type=jnp.float32)
    probs_ref[...] = logits + b2_ref[...]


def _topk_body(probs_hbm, sparse_t_hbm, idx_t_hbm, probs_v, sparse_t_v, idx_t_v):
    tpw = probs_v.shape[0]
    wid = lax.axis_index("s") * _NC + lax.axis_index("c")
    base = wid * tpw
    pltpu.sync_copy(probs_hbm.at[pl.ds(base, tpw)], probs_v)
    lanes = lax.iota(jnp.int32, 16)
    top2 = lanes < 2
    zeros16 = jnp.zeros((16,), jnp.float32)

    @plsc.parallel_loop(0, tpw, unroll=16)
    def _body(t):
        l = probs_v[t]
        e = jnp.exp(l - jnp.max(l))
        p = e / jnp.sum(e)
        sk, sv = plsc.sort_key_val(p, lanes, descending=True)
        col = lanes * 0 + t
        plsc.store_scatter(sparse_t_v, [lanes, col], zeros16)
        plsc.store_scatter(sparse_t_v, [sv, col], sk, mask=top2)
        plsc.store_scatter(idx_t_v, [lanes, col], sv, mask=top2)

    pltpu.sync_copy(sparse_t_v, sparse_t_hbm.at[:, pl.ds(base, tpw)])
    pltpu.sync_copy(idx_t_v, idx_t_hbm.at[:, pl.ds(base, tpw)])


@jax.jit
def kernel(x, W1, b1, W2, b2):
    n_tokens, hidden = x.shape
    n_mod = W2.shape[1]
    n_hid = W1.shape[1]
    probs = pl.pallas_call(
        _gate_body,
        grid=(n_tokens // _BM,),
        in_specs=[
            pl.BlockSpec((_BM, hidden), lambda i: (i, 0)),
            pl.BlockSpec((hidden, n_hid), lambda i: (0, 0)),
            pl.BlockSpec((n_hid,), lambda i: (0,)),
            pl.BlockSpec((n_hid, n_mod), lambda i: (0, 0)),
            pl.BlockSpec((n_mod,), lambda i: (0,)),
        ],
        out_specs=pl.BlockSpec((_BM, n_mod), lambda i: (i, 0)),
        out_shape=jax.ShapeDtypeStruct((n_tokens, n_mod), jnp.float32),
    )(x, W1, b1, W2, b2)

    tpw = n_tokens // _NW
    sparse_t, idx_t = pl.kernel(
        _topk_body,
        out_type=[
            jax.ShapeDtypeStruct((n_mod, n_tokens), jnp.float32),
            jax.ShapeDtypeStruct((2, n_tokens), jnp.int32),
        ],
        mesh=plsc.VectorSubcoreMesh(
            core_axis_name="c", subcore_axis_name="s",
            num_cores=_NC, num_subcores=_NS,
        ),
        compiler_params=pltpu.CompilerParams(needs_layout_passes=False),
        scratch_types=[
            pltpu.VMEM((tpw, n_mod), jnp.float32),
            pltpu.VMEM((n_mod, tpw), jnp.float32),
            pltpu.VMEM((2, tpw), jnp.int32),
        ],
    )(probs)
    return (sparse_t.T, idx_t.T)


# speedup vs baseline: 1.1386x; 1.0616x over previous
import jax
import jax.numpy as jnp
from jax import lax
from jax.experimental import pallas as pl
from jax.experimental.pallas import tpu as pltpu
from jax.experimental.pallas import tpu_sc as plsc

_BM = 1024

_NC = 2
_NS = 16
_NW = _NC * _NS


def _gate_body(x_ref, w1_ref, b1_ref, w2t_ref, b2_ref, probs_ref):
    h = jnp.dot(x_ref[...], w1_ref[...], preferred_element_type=jnp.float32)
    h = h + b1_ref[...]
    h = h * jax.nn.sigmoid(h)
    logits = lax.dot_general(
        h, w2t_ref[...], (((1,), (1,)), ((), ())),
        preferred_element_type=jnp.float32)
    probs_ref[...] = logits + b2_ref[...]


def _topk_body(probs_hbm, sparse_t_hbm, idx_t_hbm, probs_v, sparse_t_v, idx_t_v):
    tpw = probs_v.shape[0]
    wid = lax.axis_index("s") * _NC + lax.axis_index("c")
    base = wid * tpw
    pltpu.sync_copy(probs_hbm.at[pl.ds(base, tpw)], probs_v)
    lanes = lax.iota(jnp.int32, 16)
    top2 = lanes < 2
    zeros16 = jnp.zeros((16,), jnp.float32)

    @plsc.parallel_loop(0, tpw, unroll=8)
    def _body(t):
        l = probs_v[t]
        e = jnp.exp(l - jnp.max(l))
        p = e / jnp.sum(e)
        sk, sv = plsc.sort_key_val(p, lanes, descending=True)
        col = lanes * 0 + t
        plsc.store_scatter(sparse_t_v, [lanes, col], zeros16)
        plsc.store_scatter(sparse_t_v, [sv, col], sk, mask=top2)
        plsc.store_scatter(idx_t_v, [lanes, col], sv, mask=top2)

    pltpu.sync_copy(sparse_t_v, sparse_t_hbm.at[:, pl.ds(base, tpw)])
    pltpu.sync_copy(idx_t_v, idx_t_hbm.at[:, pl.ds(base, tpw)])


@jax.jit
def kernel(x, W1, b1, W2, b2):
    n_tokens, hidden = x.shape
    n_mod = W2.shape[1]
    n_hid = W1.shape[1]
    probs = pl.pallas_call(
        _gate_body,
        grid=(n_tokens // _BM,),
        in_specs=[
            pl.BlockSpec((_BM, hidden), lambda i: (i, 0)),
            pl.BlockSpec((hidden, n_hid), lambda i: (0, 0)),
            pl.BlockSpec((n_hid,), lambda i: (0,)),
            pl.BlockSpec((n_mod, n_hid), lambda i: (0, 0)),
            pl.BlockSpec((n_mod,), lambda i: (0,)),
        ],
        out_specs=pl.BlockSpec((_BM, n_mod), lambda i: (i, 0)),
        out_shape=jax.ShapeDtypeStruct((n_tokens, n_mod), jnp.float32),
    )(x, W1, b1, W2.T, b2)

    tpw = n_tokens // _NW
    sparse_t, idx_t = pl.kernel(
        _topk_body,
        out_type=[
            jax.ShapeDtypeStruct((n_mod, n_tokens), jnp.float32),
            jax.ShapeDtypeStruct((2, n_tokens), jnp.int32),
        ],
        mesh=plsc.VectorSubcoreMesh(
            core_axis_name="c", subcore_axis_name="s",
            num_cores=_NC, num_subcores=_NS,
        ),
        compiler_params=pltpu.CompilerParams(needs_layout_passes=False),
        scratch_types=[
            pltpu.VMEM((tpw, n_mod), jnp.float32),
            pltpu.VMEM((n_mod, tpw), jnp.float32),
            pltpu.VMEM((2, tpw), jnp.int32),
        ],
    )(probs)
    return (sparse_t.T, idx_t.T)


# speedup vs baseline: 1.1469x; 1.0073x over previous
import jax
import jax.numpy as jnp
from jax import lax
from jax.experimental import pallas as pl
from jax.experimental.pallas import tpu as pltpu
from jax.experimental.pallas import tpu_sc as plsc

_BM = 1024

_NC = 2
_NS = 16
_NW = _NC * _NS


def _gate_body(x_ref, w1_ref, b1_ref, w2t_ref, b2_ref, probs_ref):
    h = jnp.dot(x_ref[...], w1_ref[...], preferred_element_type=jnp.float32)
    h = h + b1_ref[...]
    h = h * jax.nn.sigmoid(h)
    logits = lax.dot_general(
        h, w2t_ref[...], (((1,), (1,)), ((), ())),
        preferred_element_type=jnp.float32)
    probs_ref[...] = logits + b2_ref[...]


def _topk_body(probs_hbm, sparse_t_hbm, idx_t_hbm, probs_v, sparse_t_v, idx_t_v):
    tpw = probs_v.shape[0]
    wid = lax.axis_index("s") * _NC + lax.axis_index("c")
    base = wid * tpw
    pltpu.sync_copy(probs_hbm.at[pl.ds(base, tpw)], probs_v)
    lanes = lax.iota(jnp.int32, 16)
    top2 = lanes < 2
    zeros16 = jnp.zeros((16,), jnp.float32)

    @plsc.parallel_loop(0, tpw, unroll=4)
    def _body(t):
        l = probs_v[t]
        e = jnp.exp(l - jnp.max(l))
        p = e / jnp.sum(e)
        sk, sv = plsc.sort_key_val(p, lanes, descending=True)
        col = lanes * 0 + t
        plsc.store_scatter(sparse_t_v, [lanes, col], zeros16)
        plsc.store_scatter(sparse_t_v, [sv, col], sk, mask=top2)
        plsc.store_scatter(idx_t_v, [lanes, col], sv, mask=top2)

    pltpu.sync_copy(sparse_t_v, sparse_t_hbm.at[:, pl.ds(base, tpw)])
    pltpu.sync_copy(idx_t_v, idx_t_hbm.at[:, pl.ds(base, tpw)])


@jax.jit
def kernel(x, W1, b1, W2, b2):
    n_tokens, hidden = x.shape
    n_mod = W2.shape[1]
    n_hid = W1.shape[1]
    probs = pl.pallas_call(
        _gate_body,
        grid=(n_tokens // _BM,),
        in_specs=[
            pl.BlockSpec((_BM, hidden), lambda i: (i, 0)),
            pl.BlockSpec((hidden, n_hid), lambda i: (0, 0)),
            pl.BlockSpec((n_hid,), lambda i: (0,)),
            pl.BlockSpec((n_mod, n_hid), lambda i: (0, 0)),
            pl.BlockSpec((n_mod,), lambda i: (0,)),
        ],
        out_specs=pl.BlockSpec((_BM, n_mod), lambda i: (i, 0)),
        out_shape=jax.ShapeDtypeStruct((n_tokens, n_mod), jnp.float32),
    )(x, W1, b1, W2.T, b2)

    tpw = n_tokens // _NW
    sparse_t, idx_t = pl.kernel(
        _topk_body,
        out_type=[
            jax.ShapeDtypeStruct((n_mod, n_tokens), jnp.float32),
            jax.ShapeDtypeStruct((2, n_tokens), jnp.int32),
        ],
        mesh=plsc.VectorSubcoreMesh(
            core_axis_name="c", subcore_axis_name="s",
            num_cores=_NC, num_subcores=_NS,
        ),
        compiler_params=pltpu.CompilerParams(needs_layout_passes=False),
        scratch_types=[
            pltpu.VMEM((tpw, n_mod), jnp.float32),
            pltpu.VMEM((n_mod, tpw), jnp.float32),
            pltpu.VMEM((2, tpw), jnp.int32),
        ],
    )(probs)
    return (sparse_t.T, idx_t.T)
